# baseline (speedup 1.0000x reference)
import functools

import numpy as _np

import jax
import jax.numpy as jnp
from jax import lax
from jax.experimental import pallas as pl
from jax.experimental.pallas import tpu as pltpu
from jax.experimental.pallas import tpu_sc as plsc

N = 10000
NP = 10240
E = 320000
CH = 125
CPT = 80
SEG = 2
SCPT = CPT // SEG
RPT = NP // 16
NBUF = 8
NBUF2 = 2

_mesh = plsc.VectorSubcoreMesh(core_axis_name="c", subcore_axis_name="s")


_sc_params = pltpu.CompilerParams(use_tc_tiling_on_sc=False)


def _sc_di_body(xdp, edi, eid, z32, ones32,
                sum_di_out,
                sdi_v, ddi_v, did_v, rows0, rows1, rows2, rows3,
                rows4, rows5, rows6, rows7, ones_v, acc,
                semg0, semg1, semg2, semg3, semg4, semg5, semg6, semg7,
                sems0, sems1, sems2, sems3, sems4, sems5, sems6, sems7):
    rows = (rows0, rows1, rows2, rows3, rows4, rows5, rows6, rows7)
    semg = (semg0, semg1, semg2, semg3, semg4, semg5, semg6, semg7)
    sems = (sems0, sems1, sems2, sems3, sems4, sems5, sems6, sems7)
    cid = lax.axis_index("c")
    tid = lax.axis_index("s")
    r0 = pl.multiple_of(tid * RPT, 8)
    pltpu.sync_copy(z32.at[pl.ds(r0, RPT)], acc.at[pl.ds(r0, RPT)])
    pltpu.sync_copy(ones32, ones_v)
    cb = pl.multiple_of(cid * (16 * CPT) + tid * CPT, 8)
    pltpu.sync_copy(edi.at[0, pl.ds(cb, CPT)], sdi_v)
    pltpu.sync_copy(edi.at[1, pl.ds(cb, CPT)], ddi_v)
    pltpu.sync_copy(eid.at[1, pl.ds(cb, CPT)], did_v)
    for b in range(NBUF - 1):
        pltpu.async_copy(xdp.at[sdi_v.at[b]], rows[b], semg[b])
    plsc.subcore_barrier()

    def body(k, carry):
        for b in range(NBUF):
            j = k * NBUF + b
            bp = (b + NBUF - 1) % NBUF
            pltpu.make_async_copy(xdp.at[sdi_v.at[j]], rows[b],
                                  semg[b]).wait()
            pltpu.async_copy(rows[b], acc.at[ddi_v.at[j]], sems[b], add=True)
            pltpu.async_copy(ones_v, acc.at[did_v.at[j]], sems[b], add=True)

            @pl.when(j >= 1)
            def _():
                pltpu.make_async_copy(rows[bp], acc.at[ddi_v.at[j - 1]],
                                      sems[bp]).wait()
                pltpu.make_async_copy(ones_v, acc.at[did_v.at[j - 1]],
                                      sems[bp]).wait()

            @pl.when(j + NBUF - 1 < CPT)
            def _():
                pltpu.async_copy(xdp.at[sdi_v.at[j + NBUF - 1]], rows[bp],
                                 semg[bp])
        return carry

    lax.fori_loop(0, CPT // NBUF, body, 0)
    bl = (CPT - 1) % NBUF
    pltpu.make_async_copy(rows[bl], acc.at[ddi_v.at[CPT - 1]],
                          sems[bl]).wait()
    pltpu.make_async_copy(ones_v, acc.at[did_v.at[CPT - 1]],
                          sems[bl]).wait()
    plsc.subcore_barrier()
    pltpu.sync_copy(acc.at[pl.ds(r0, RPT)],
                    sum_di_out.at[cid, pl.ds(r0, RPT)])


_sc_di = functools.partial(
    pl.kernel, _sc_di_body, mesh=_mesh,
    compiler_params=_sc_params,
    out_type=jax.ShapeDtypeStruct((2, NP, 32), jnp.float32),
    scratch_types=(
        [pltpu.VMEM((CPT, CH), jnp.int32)] * 3
        + [pltpu.VMEM((CH, 32), jnp.float32)] * (NBUF + 1)
        + [pltpu.VMEM_SHARED((NP, 32), jnp.float32)]
        + [pltpu.SemaphoreType.DMA] * (2 * NBUF)
    ),
)()


def _sc_seg128_body(feat, eid, z128,
                    sum_out,
                    sid_v, did_v, rows0, rows1, acc,
                    semg0, semg1, sems0, sems1):
    rows = (rows0, rows1)
    semg = (semg0, semg1)
    sems = (sems0, sems1)
    cid = lax.axis_index("c")
    tid = lax.axis_index("s")
    r0 = pl.multiple_of(tid * RPT, 8)
    pltpu.sync_copy(z128.at[pl.ds(r0, RPT)], acc.at[pl.ds(r0, RPT)])
    for seg in range(SEG):
        cbs = pl.multiple_of(cid * (16 * CPT) + tid * CPT + seg * SCPT, 8)
        pltpu.sync_copy(eid.at[0, pl.ds(cbs, SCPT)], sid_v)
        pltpu.sync_copy(eid.at[1, pl.ds(cbs, SCPT)], did_v)
        for b in range(NBUF2 - 1):
            pltpu.async_copy(feat.at[sid_v.at[b]], rows[b], semg[b])
        if seg == 0:
            plsc.subcore_barrier()

        def body(k, carry):
            for b in range(NBUF2):
                j = k * NBUF2 + b
                bp = (b + 1) % NBUF2
                pltpu.make_async_copy(feat.at[sid_v.at[j]], rows[b],
                                      semg[b]).wait()
                pltpu.async_copy(rows[b], acc.at[did_v.at[j]], sems[b],
                                 add=True)

                @pl.when(j >= 1)
                def _():
                    pltpu.make_async_copy(rows[bp], acc.at[did_v.at[j - 1]],
                                          sems[bp]).wait()

                @pl.when(j + NBUF2 - 1 < SCPT)
                def _():
                    pltpu.async_copy(feat.at[sid_v.at[j + NBUF2 - 1]],
                                     rows[bp], semg[bp])
            return carry

        lax.fori_loop(0, SCPT // NBUF2, body, 0)
        bl = (SCPT - 1) % NBUF2
        pltpu.make_async_copy(rows[bl], acc.at[did_v.at[SCPT - 1]],
                              sems[bl]).wait()
    plsc.subcore_barrier()
    pltpu.sync_copy(acc.at[pl.ds(r0, RPT)], sum_out.at[cid, pl.ds(r0, RPT)])


_sc_seg128 = functools.partial(
    pl.kernel, _sc_seg128_body, mesh=_mesh,
    compiler_params=_sc_params,
    out_type=jax.ShapeDtypeStruct((2, NP, 128), jnp.float32),
    scratch_types=[
        pltpu.VMEM((SCPT, CH), jnp.int32),
        pltpu.VMEM((SCPT, CH), jnp.int32),
        pltpu.VMEM((CH, 128), jnp.float32),
        pltpu.VMEM((CH, 128), jnp.float32),
        pltpu.VMEM_SHARED((NP, 128), jnp.float32),
        pltpu.SemaphoreType.DMA,
        pltpu.SemaphoreType.DMA,
        pltpu.SemaphoreType.DMA,
        pltpu.SemaphoreType.DMA,
    ],
)()


_BR = 1000


def _leaky(x):
    return jnp.where(x >= 0.0, x, 0.01 * x)


def _tc_mlp_body(xd_ref, w1_ref, b1_ref, w2_ref, b2_ref, out_ref):
    x = xd_ref[...]
    h = jnp.maximum(jnp.dot(x[:, :128], w1_ref[...],
                            preferred_element_type=jnp.float32) + b1_ref[...],
                    0.0)
    mlp = jnp.dot(h, w2_ref[...], preferred_element_type=jnp.float32) + b2_ref[...]
    ones = jnp.ones((_BR, 1), jnp.float32)
    zer = jnp.zeros((_BR, 11), jnp.float32)
    out_ref[...] = jnp.concatenate([mlp, x[:, 128:138], ones, zer], axis=1)


def _tc_l0i_body(sdia_ref, sdib_ref, xi_ref, wldi_ref, bldi_ref, wrdi_ref,
                 xi1_ref):
    sum_di = sdia_ref[0] + sdib_ref[0]
    cnt_di = jnp.maximum(sum_di[:, 20:21], 1.0)
    aggr_i = sum_di[:, :20] / cnt_di
    xi = xi_ref[...]
    xi1 = (jnp.dot(aggr_i, wldi_ref[...], preferred_element_type=jnp.float32)
           + bldi_ref[...]
           + jnp.dot(xi, wrdi_ref[...], preferred_element_type=jnp.float32))
    xi1_ref[...] = _leaky(xi1)


def _tc_l0d_body(sdia_ref, sdib_ref, sida_ref, sidb_ref,
                 xdp_ref, wlid_ref, blid_ref, wrid_ref, xd1_ref):
    sum_di = sdia_ref[0] + sdib_ref[0]
    sum_id = sida_ref[0] + sidb_ref[0]
    cnt_id = jnp.maximum(sum_di[:, 24:25], 1.0)
    aggr_d = sum_id / cnt_id
    xdp = xdp_ref[...]
    xd0 = xdp[:, :20]
    res = xdp[:, 10:20]
    t = (jnp.dot(aggr_d, wlid_ref[...], preferred_element_type=jnp.float32)
         + blid_ref[...]
         + jnp.dot(xd0, wrid_ref[...], preferred_element_type=jnp.float32))
    t = jnp.concatenate([t[:, :118], t[:, 118:] + res], axis=1)
    xd1_ref[...] = _leaky(t)


def _tc_l1_body(suma_ref, sumb_ref, cnta_ref, cntb_ref, xd1_ref,
                wlid_ref, blid_ref, wrid_ref, w0_ref, b0_ref, wf_ref, bf_ref,
                out_ref):
    cnt = jnp.maximum(cnta_ref[0][:, 24:25] + cntb_ref[0][:, 24:25], 1.0)
    aggr = (suma_ref[0] + sumb_ref[0]) / cnt
    xd1 = xd1_ref[...]
    t = (jnp.dot(aggr, wlid_ref[...], preferred_element_type=jnp.float32)
         + blid_ref[...]
         + jnp.dot(xd1, wrid_ref[...], preferred_element_type=jnp.float32))
    t = jnp.concatenate([t[:, :118], t[:, 118:] + xd1[:, 118:]], axis=1)
    xd2 = _leaky(t)
    h = jnp.maximum(jnp.dot(xd2, w0_ref[...],
                            preferred_element_type=jnp.float32) + b0_ref[...],
                    0.0)
    out_ref[...] = jnp.dot(h, wf_ref[...],
                           preferred_element_type=jnp.float32) + bf_ref[...]


def _row_spec(d):
    return pl.BlockSpec((_BR, d), lambda i: (i, 0))


def _part_spec(d, h):
    return pl.BlockSpec((1, _BR, d), lambda i, _h=h: (_h, i, 0))


def _full_spec(shape):
    return pl.BlockSpec(shape, lambda i: (0,) * len(shape))


def kernel(x_drivers, x_items, edge_index_di, edge_index_id, W1, b1, W2, b2,
           Wl0_di, bl0_di, Wr0_di, Wl0_id, bl0_id, Wr0_id, Wl1_di, bl1_di,
           Wr1_di, Wl1_id, bl1_id, Wr1_id, W0, b0, Wf, bf):
    f32 = jnp.float32
    xd_pad = pl.pallas_call(
        _tc_mlp_body,
        grid=(N // _BR,),
        in_specs=[_row_spec(138), _full_spec((128, 20)), _full_spec((1, 20)),
                  _full_spec((20, 10)), _full_spec((1, 10))],
        out_specs=_row_spec(32),
        out_shape=jax.ShapeDtypeStruct((N, 32), f32),
    )(x_drivers, W1, b1.reshape(1, -1), W2, b2.reshape(1, -1))

    edi = edge_index_di.reshape(2, E // CH, CH)
    eid = edge_index_id.reshape(2, E // CH, CH)
    z32 = jnp.asarray(_np.zeros((NP, 32), _np.float32))
    z128 = jnp.asarray(_np.zeros((NP, 128), _np.float32))
    ones32 = jnp.asarray(
        _np.concatenate([_np.zeros((CH, 24), _np.float32),
                         _np.ones((CH, 8), _np.float32)], axis=1))

    sum_di_p = _sc_di(xd_pad, edi, eid, z32, ones32)
    sum_id_p = _sc_seg128(x_items, eid, z128)

    xi1 = pl.pallas_call(
        _tc_l0i_body,
        grid=(N // _BR,),
        in_specs=[_part_spec(32, 0), _part_spec(32, 1),
                  _row_spec(128),
                  _full_spec((20, 128)), _full_spec((1, 128)),
                  _full_spec((128, 128))],
        out_specs=_row_spec(128),
        out_shape=jax.ShapeDtypeStruct((N, 128), f32),
    )(sum_di_p, sum_di_p, x_items, Wl0_di, bl0_di.reshape(1, -1), Wr0_di)

    sum_id1_p = _sc_seg128(xi1, eid, z128)

    xd1 = pl.pallas_call(
        _tc_l0d_body,
        grid=(N // _BR,),
        in_specs=[_part_spec(32, 0), _part_spec(32, 1),
                  _part_spec(128, 0), _part_spec(128, 1),
                  _row_spec(32),
                  _full_spec((128, 128)), _full_spec((1, 128)),
                  _full_spec((20, 128))],
        out_specs=_row_spec(128),
        out_shape=jax.ShapeDtypeStruct((N, 128), f32),
    )(sum_di_p, sum_di_p, sum_id_p, sum_id_p, xd_pad,
      Wl0_id, bl0_id.reshape(1, -1), Wr0_id)

    out = pl.pallas_call(
        _tc_l1_body,
        grid=(N // _BR,),
        in_specs=[_part_spec(128, 0), _part_spec(128, 1),
                  _part_spec(32, 0), _part_spec(32, 1),
                  _row_spec(128),
                  _full_spec((128, 128)), _full_spec((1, 128)),
                  _full_spec((128, 128)), _full_spec((128, 64)),
                  _full_spec((1, 64)), _full_spec((64, 1)),
                  _full_spec((1, 1))],
        out_specs=_row_spec(1),
        out_shape=jax.ShapeDtypeStruct((N, 1), f32),
    )(sum_id1_p, sum_id1_p, sum_di_p, sum_di_p, xd1,
      Wl1_id, bl1_id.reshape(1, -1), Wr1_id, W0, b0.reshape(1, -1),
      Wf, bf.reshape(1, -1))
    return out


# speedup vs baseline: 1.1181x; 1.1181x over previous
import functools

import numpy as _np

import jax
import jax.numpy as jnp
from jax import lax
from jax.experimental import pallas as pl
from jax.experimental.pallas import tpu as pltpu
from jax.experimental.pallas import tpu_sc as plsc

N = 10000
NP = 10240
E = 320000
CH = 125
CPT = 80
SEG = 2
SCPT = CPT // SEG
RPT = NP // 16
NBUF = 8
NBUF2 = 2

_mesh = plsc.VectorSubcoreMesh(core_axis_name="c", subcore_axis_name="s")


_sc_params = pltpu.CompilerParams(use_tc_tiling_on_sc=False)


def _sc_di_body(xdp, edi, eid, z32, ones32,
                sum_di_out,
                sdi_v, ddi_v, did_v, rows0, rows1, rows2, rows3,
                rows4, rows5, rows6, rows7, ones_v, acc,
                semg0, semg1, semg2, semg3, semg4, semg5, semg6, semg7,
                sems0, sems1, sems2, sems3, sems4, sems5, sems6, sems7):
    rows = (rows0, rows1, rows2, rows3, rows4, rows5, rows6, rows7)
    semg = (semg0, semg1, semg2, semg3, semg4, semg5, semg6, semg7)
    sems = (sems0, sems1, sems2, sems3, sems4, sems5, sems6, sems7)
    cid = lax.axis_index("c")
    tid = lax.axis_index("s")
    r0 = pl.multiple_of(tid * RPT, 8)
    pltpu.sync_copy(z32.at[pl.ds(r0, RPT)], acc.at[pl.ds(r0, RPT)])
    pltpu.sync_copy(ones32, ones_v)
    cb = pl.multiple_of(cid * (16 * CPT) + tid * CPT, 8)
    pltpu.sync_copy(edi.at[0, pl.ds(cb, CPT)], sdi_v)
    pltpu.sync_copy(edi.at[1, pl.ds(cb, CPT)], ddi_v)
    pltpu.sync_copy(eid.at[1, pl.ds(cb, CPT)], did_v)
    for b in range(NBUF - 1):
        pltpu.async_copy(xdp.at[sdi_v.at[b]], rows[b], semg[b])
    plsc.subcore_barrier()

    def body(k, carry):
        for b in range(NBUF):
            j = k * NBUF + b
            bp = (b + NBUF - 1) % NBUF
            pltpu.make_async_copy(xdp.at[sdi_v.at[j]], rows[b],
                                  semg[b]).wait()
            pltpu.async_copy(rows[b], acc.at[ddi_v.at[j]], sems[b], add=True)
            pltpu.async_copy(ones_v, acc.at[did_v.at[j]], sems[b], add=True)

            @pl.when(j >= 1)
            def _():
                pltpu.make_async_copy(rows[bp], acc.at[ddi_v.at[j - 1]],
                                      sems[bp]).wait()
                pltpu.make_async_copy(ones_v, acc.at[did_v.at[j - 1]],
                                      sems[bp]).wait()

            @pl.when(j + NBUF - 1 < CPT)
            def _():
                pltpu.async_copy(xdp.at[sdi_v.at[j + NBUF - 1]], rows[bp],
                                 semg[bp])
        return carry

    lax.fori_loop(0, CPT // NBUF, body, 0)
    bl = (CPT - 1) % NBUF
    pltpu.make_async_copy(rows[bl], acc.at[ddi_v.at[CPT - 1]],
                          sems[bl]).wait()
    pltpu.make_async_copy(ones_v, acc.at[did_v.at[CPT - 1]],
                          sems[bl]).wait()
    plsc.subcore_barrier()
    pltpu.sync_copy(acc.at[pl.ds(r0, RPT)],
                    sum_di_out.at[cid, pl.ds(r0, RPT)])


_sc_di = functools.partial(
    pl.kernel, _sc_di_body, mesh=_mesh,
    compiler_params=_sc_params,
    out_type=jax.ShapeDtypeStruct((2, NP, 32), jnp.float32),
    scratch_types=(
        [pltpu.VMEM((CPT, CH), jnp.int32)] * 3
        + [pltpu.VMEM((CH, 32), jnp.float32)] * (NBUF + 1)
        + [pltpu.VMEM_SHARED((NP, 32), jnp.float32)]
        + [pltpu.SemaphoreType.DMA] * (2 * NBUF)
    ),
)()


def _sc_seg128_body(feat, eid, z128,
                    sum_out,
                    sid_v, did_v, rows0, rows1, acc,
                    semg0, semg1, sems0, sems1):
    rows = (rows0, rows1)
    semg = (semg0, semg1)
    sems = (sems0, sems1)
    cid = lax.axis_index("c")
    tid = lax.axis_index("s")
    r0 = pl.multiple_of(tid * RPT, 8)
    pltpu.sync_copy(z128.at[pl.ds(r0, RPT)], acc.at[pl.ds(r0, RPT)])
    for seg in range(SEG):
        cbs = pl.multiple_of(cid * (16 * CPT) + tid * CPT + seg * SCPT, 8)
        pltpu.sync_copy(eid.at[0, pl.ds(cbs, SCPT)], sid_v)
        pltpu.sync_copy(eid.at[1, pl.ds(cbs, SCPT)], did_v)
        for b in range(NBUF2):
            pltpu.async_copy(feat.at[sid_v.at[b]], rows[b], semg[b])
        if seg == 0:
            plsc.subcore_barrier()

        def body(k, carry):
            for b in range(NBUF2):
                j = k * NBUF2 + b
                pltpu.make_async_copy(feat.at[sid_v.at[j]], rows[b],
                                      semg[b]).wait()
                pltpu.sync_copy(rows[b], acc.at[did_v.at[j]], add=True)

                @pl.when(j + NBUF2 < SCPT)
                def _():
                    pltpu.async_copy(feat.at[sid_v.at[j + NBUF2]], rows[b],
                                     semg[b])
            return carry

        lax.fori_loop(0, SCPT // NBUF2, body, 0)
    plsc.subcore_barrier()
    pltpu.sync_copy(acc.at[pl.ds(r0, RPT)], sum_out.at[cid, pl.ds(r0, RPT)])


_sc_seg128 = functools.partial(
    pl.kernel, _sc_seg128_body, mesh=_mesh,
    compiler_params=_sc_params,
    out_type=jax.ShapeDtypeStruct((2, NP, 128), jnp.float32),
    scratch_types=[
        pltpu.VMEM((SCPT, CH), jnp.int32),
        pltpu.VMEM((SCPT, CH), jnp.int32),
        pltpu.VMEM((CH, 128), jnp.float32),
        pltpu.VMEM((CH, 128), jnp.float32),
        pltpu.VMEM_SHARED((NP, 128), jnp.float32),
        pltpu.SemaphoreType.DMA,
        pltpu.SemaphoreType.DMA,
        pltpu.SemaphoreType.DMA,
        pltpu.SemaphoreType.DMA,
    ],
)()


_BR = 1000


def _leaky(x):
    return jnp.where(x >= 0.0, x, 0.01 * x)


def _tc_mlp_body(xd_ref, w1_ref, b1_ref, w2_ref, b2_ref, out_ref):
    x = xd_ref[...]
    h = jnp.maximum(jnp.dot(x[:, :128], w1_ref[...],
                            preferred_element_type=jnp.float32) + b1_ref[...],
                    0.0)
    mlp = jnp.dot(h, w2_ref[...], preferred_element_type=jnp.float32) + b2_ref[...]
    ones = jnp.ones((_BR, 1), jnp.float32)
    zer = jnp.zeros((_BR, 11), jnp.float32)
    out_ref[...] = jnp.concatenate([mlp, x[:, 128:138], ones, zer], axis=1)


def _tc_l0i_body(sdia_ref, sdib_ref, xi_ref, wldi_ref, bldi_ref, wrdi_ref,
                 xi1_ref):
    sum_di = sdia_ref[0] + sdib_ref[0]
    cnt_di = jnp.maximum(sum_di[:, 20:21], 1.0)
    aggr_i = sum_di[:, :20] / cnt_di
    xi = xi_ref[...]
    xi1 = (jnp.dot(aggr_i, wldi_ref[...], preferred_element_type=jnp.float32)
           + bldi_ref[...]
           + jnp.dot(xi, wrdi_ref[...], preferred_element_type=jnp.float32))
    xi1_ref[...] = _leaky(xi1)


def _tc_l0d_body(sdia_ref, sdib_ref, sida_ref, sidb_ref,
                 xdp_ref, wlid_ref, blid_ref, wrid_ref, xd1_ref):
    sum_di = sdia_ref[0] + sdib_ref[0]
    sum_id = sida_ref[0] + sidb_ref[0]
    cnt_id = jnp.maximum(sum_di[:, 24:25], 1.0)
    aggr_d = sum_id / cnt_id
    xdp = xdp_ref[...]
    xd0 = xdp[:, :20]
    res = xdp[:, 10:20]
    t = (jnp.dot(aggr_d, wlid_ref[...], preferred_element_type=jnp.float32)
         + blid_ref[...]
         + jnp.dot(xd0, wrid_ref[...], preferred_element_type=jnp.float32))
    t = jnp.concatenate([t[:, :118], t[:, 118:] + res], axis=1)
    xd1_ref[...] = _leaky(t)


def _tc_l1_body(suma_ref, sumb_ref, cnta_ref, cntb_ref, xd1_ref,
                wlid_ref, blid_ref, wrid_ref, w0_ref, b0_ref, wf_ref, bf_ref,
                out_ref):
    cnt = jnp.maximum(cnta_ref[0][:, 24:25] + cntb_ref[0][:, 24:25], 1.0)
    aggr = (suma_ref[0] + sumb_ref[0]) / cnt
    xd1 = xd1_ref[...]
    t = (jnp.dot(aggr, wlid_ref[...], preferred_element_type=jnp.float32)
         + blid_ref[...]
         + jnp.dot(xd1, wrid_ref[...], preferred_element_type=jnp.float32))
    t = jnp.concatenate([t[:, :118], t[:, 118:] + xd1[:, 118:]], axis=1)
    xd2 = _leaky(t)
    h = jnp.maximum(jnp.dot(xd2, w0_ref[...],
                            preferred_element_type=jnp.float32) + b0_ref[...],
                    0.0)
    out_ref[...] = jnp.dot(h, wf_ref[...],
                           preferred_element_type=jnp.float32) + bf_ref[...]


def _row_spec(d):
    return pl.BlockSpec((_BR, d), lambda i: (i, 0))


def _part_spec(d, h):
    return pl.BlockSpec((1, _BR, d), lambda i, _h=h: (_h, i, 0))


def _full_spec(shape):
    return pl.BlockSpec(shape, lambda i: (0,) * len(shape))


def kernel(x_drivers, x_items, edge_index_di, edge_index_id, W1, b1, W2, b2,
           Wl0_di, bl0_di, Wr0_di, Wl0_id, bl0_id, Wr0_id, Wl1_di, bl1_di,
           Wr1_di, Wl1_id, bl1_id, Wr1_id, W0, b0, Wf, bf):
    f32 = jnp.float32
    xd_pad = pl.pallas_call(
        _tc_mlp_body,
        grid=(N // _BR,),
        in_specs=[_row_spec(138), _full_spec((128, 20)), _full_spec((1, 20)),
                  _full_spec((20, 10)), _full_spec((1, 10))],
        out_specs=_row_spec(32),
        out_shape=jax.ShapeDtypeStruct((N, 32), f32),
    )(x_drivers, W1, b1.reshape(1, -1), W2, b2.reshape(1, -1))

    edi = edge_index_di.reshape(2, E // CH, CH)
    eid = edge_index_id.reshape(2, E // CH, CH)
    z32 = jnp.asarray(_np.zeros((NP, 32), _np.float32))
    z128 = jnp.asarray(_np.zeros((NP, 128), _np.float32))
    ones32 = jnp.asarray(
        _np.concatenate([_np.zeros((CH, 24), _np.float32),
                         _np.ones((CH, 8), _np.float32)], axis=1))

    sum_di_p = _sc_di(xd_pad, edi, eid, z32, ones32)
    sum_id_p = _sc_seg128(x_items, eid, z128)

    xi1 = pl.pallas_call(
        _tc_l0i_body,
        grid=(N // _BR,),
        in_specs=[_part_spec(32, 0), _part_spec(32, 1),
                  _row_spec(128),
                  _full_spec((20, 128)), _full_spec((1, 128)),
                  _full_spec((128, 128))],
        out_specs=_row_spec(128),
        out_shape=jax.ShapeDtypeStruct((N, 128), f32),
    )(sum_di_p, sum_di_p, x_items, Wl0_di, bl0_di.reshape(1, -1), Wr0_di)

    sum_id1_p = _sc_seg128(xi1, eid, z128)

    xd1 = pl.pallas_call(
        _tc_l0d_body,
        grid=(N // _BR,),
        in_specs=[_part_spec(32, 0), _part_spec(32, 1),
                  _part_spec(128, 0), _part_spec(128, 1),
                  _row_spec(32),
                  _full_spec((128, 128)), _full_spec((1, 128)),
                  _full_spec((20, 128))],
        out_specs=_row_spec(128),
        out_shape=jax.ShapeDtypeStruct((N, 128), f32),
    )(sum_di_p, sum_di_p, sum_id_p, sum_id_p, xd_pad,
      Wl0_id, bl0_id.reshape(1, -1), Wr0_id)

    out = pl.pallas_call(
        _tc_l1_body,
        grid=(N // _BR,),
        in_specs=[_part_spec(128, 0), _part_spec(128, 1),
                  _part_spec(32, 0), _part_spec(32, 1),
                  _row_spec(128),
                  _full_spec((128, 128)), _full_spec((1, 128)),
                  _full_spec((128, 128)), _full_spec((128, 64)),
                  _full_spec((1, 64)), _full_spec((64, 1)),
                  _full_spec((1, 1))],
        out_specs=_row_spec(1),
        out_shape=jax.ShapeDtypeStruct((N, 1), f32),
    )(sum_id1_p, sum_id1_p, sum_di_p, sum_di_p, xd1,
      Wl1_id, bl1_id.reshape(1, -1), Wr1_id, W0, b0.reshape(1, -1),
      Wf, bf.reshape(1, -1))
    return out
